# baseline (speedup 1.0000x reference)
import jax
import jax.numpy as jnp
from jax import lax
from jax.experimental import pallas as pl
from jax.experimental.pallas import tpu as pltpu
from jax.experimental.pallas import tpu_sc as plsc

EMB = 64
SEQ = 200
BATCH = 4096
NW = 32
BLK = 128
NBUF = 4
NLANE = 16
SCALE = 8.0
KE = EMB // NLANE
OPAD = BLK + 1


def _body(ids_hbm, tok_hbm, pos_hbm, out_hbm, idx_v, pos_v, gbuf, obuf,
          gs0, gs1, gs2, gs3, os0, os1, os2, os3):
    gsems = [gs0, gs1, gs2, gs3]
    osems = [os0, os1, os2, os3]
    cid = lax.axis_index("c")
    sid = lax.axis_index("s")
    wid = cid * 16 + sid
    b0 = wid * BLK
    pltpu.sync_copy(ids_hbm.at[:, pl.ds(b0, BLK)], idx_v)
    pltpu.sync_copy(pos_hbm, pos_v)

    iot = lax.iota(jnp.int32, NLANE)
    eslot = [[iot + NLANE * k + EMB * slot for k in range(KE)]
             for slot in range(NBUF)]

    def gather_copy(slot):
        return pltpu.make_async_copy(
            tok_hbm.at[idx_v.at[0]], gbuf.at[slot], gsems[slot])

    def gather_start(s, slot):
        pltpu.make_async_copy(
            tok_hbm.at[idx_v.at[s]], gbuf.at[slot], gsems[slot]).start()

    def out_copies(s, slot):
        return [pltpu.make_async_copy(
            obuf.at[pl.ds(slot * EMB + 8 * r, 8), pl.ds(0, BLK)],
            out_hbm.at[s, r, wid], osems[slot])
            for r in range(8)]

    def out_wait(slot):
        pltpu.make_async_copy(
            tok_hbm.at[pl.ds(0, BLK)], gbuf.at[slot], osems[slot]).wait()

    def compute_chunk(s, slot):
        pvec = [pos_v[s, pl.ds(NLANE * k, NLANE)] for k in range(KE)]
        RB = 8

        def group(g, c):
            j0 = g * RB
            chains = [(jl, k) for jl in range(RB) for k in range(KE)]
            gvs = [gbuf[slot, j0 + jl, pl.ds(NLANE * k, NLANE)]
                   for jl, k in chains]
            jvec = [jnp.zeros((NLANE,), jnp.int32) + (j0 + jl)
                    for jl in range(RB)]
            vals = [SCALE * gv for gv in gvs]
            vals = [v + pvec[k] for v, (_, k) in zip(vals, chains)]
            for (jl, k), val in zip(chains, vals):
                plsc.store_scatter(obuf, [eslot[slot][k], jvec[jl]], val)
            return c

        lax.fori_loop(0, BLK // RB, group, 0)

    for b in range(NBUF):
        gather_start(b, b)

    def outer(i0, carry):
        for b in range(NBUF):
            s = i0 * NBUF + b
            gather_copy(b).wait()

            @pl.when(i0 >= 1)
            def _():
                out_wait(b)

            compute_chunk(s, b)
            for c in out_copies(s, b):
                c.start()

            @pl.when(s + NBUF < SEQ)
            def _():
                gather_start(s + NBUF, b)
        return carry

    lax.fori_loop(0, SEQ // NBUF, outer, 0)
    for t in range(SEQ - SEQ % NBUF, SEQ):
        b = t % NBUF
        gather_copy(b).wait()
        out_wait(b)
        compute_chunk(t, b)
        for c in out_copies(t, b):
            c.start()
    for t in range(SEQ - NBUF, SEQ):
        out_wait(t % NBUF)


def kernel(input_ids, tok_table, pos_table):
    ids_t = input_ids.astype(jnp.int32).T
    mesh = plsc.VectorSubcoreMesh(core_axis_name="c", subcore_axis_name="s")
    out = pl.kernel(
        _body,
        out_type=jax.ShapeDtypeStruct((SEQ, 8, NW, 8, BLK), jnp.float32),
        mesh=mesh,
        compiler_params=pltpu.CompilerParams(use_tc_tiling_on_sc=False,
                                             needs_layout_passes=False),
        scratch_types=[
            pltpu.VMEM((SEQ, BLK), jnp.int32),
            pltpu.VMEM((SEQ, EMB), jnp.float32),
            pltpu.VMEM((NBUF, BLK, EMB), jnp.float32),
            pltpu.VMEM((NBUF * EMB, OPAD), jnp.float32),
        ] + [pltpu.SemaphoreType.DMA] * (2 * NBUF),
    )(ids_t, tok_table, pos_table)
    out = out.transpose(2, 4, 0, 1, 3)
    return out.reshape(BATCH, SEQ, EMB)


# speedup vs baseline: 1.0153x; 1.0153x over previous
import jax
import jax.numpy as jnp
from jax import lax
from jax.experimental import pallas as pl
from jax.experimental.pallas import tpu as pltpu
from jax.experimental.pallas import tpu_sc as plsc

EMB = 64
SEQ = 200
BATCH = 4096
NW = 32
BLK = 128
NBUF = 4
NLANE = 16
SCALE = 8.0
KE = EMB // NLANE
OPAD = BLK + 1


def _body(ids_hbm, tok_hbm, pos_hbm, out_hbm, idx_v, pos_v, gbuf, obuf,
          gs0, gs1, gs2, gs3, os0, os1, os2, os3):
    gsems = [gs0, gs1, gs2, gs3]
    osems = [os0, os1, os2, os3]
    cid = lax.axis_index("c")
    sid = lax.axis_index("s")
    wid = cid * 16 + sid
    b0 = wid * BLK
    pltpu.sync_copy(ids_hbm.at[:, pl.ds(b0, BLK)], idx_v)
    pltpu.sync_copy(pos_hbm, pos_v)

    iot = lax.iota(jnp.int32, NLANE)
    eslot = [[iot + NLANE * k + EMB * slot for k in range(KE)]
             for slot in range(NBUF)]

    def gather_copy(slot):
        return pltpu.make_async_copy(
            tok_hbm.at[idx_v.at[0]], gbuf.at[slot], gsems[slot])

    def gather_start(s, slot):
        pltpu.make_async_copy(
            tok_hbm.at[idx_v.at[s]], gbuf.at[slot], gsems[slot]).start()

    def out_copies(s, slot):
        return [pltpu.make_async_copy(
            obuf.at[pl.ds(slot * EMB + 8 * r, 8), pl.ds(0, BLK)],
            out_hbm.at[s, r, wid], osems[slot])
            for r in range(8)]

    def out_wait(slot):
        pltpu.make_async_copy(
            tok_hbm.at[pl.ds(0, BLK)], gbuf.at[slot], osems[slot]).wait()

    def compute_chunk(s, slot):
        pvec = [pos_v[s, pl.ds(NLANE * k, NLANE)] for k in range(KE)]
        RB = 4

        def group(g, c):
            j0 = g * RB
            chains = [(jl, k) for jl in range(RB) for k in range(KE)]
            gvs = [gbuf[slot, j0 + jl, pl.ds(NLANE * k, NLANE)]
                   for jl, k in chains]
            jvec = [jnp.zeros((NLANE,), jnp.int32) + (j0 + jl)
                    for jl in range(RB)]
            vals = [SCALE * gv for gv in gvs]
            vals = [v + pvec[k] for v, (_, k) in zip(vals, chains)]
            for (jl, k), val in zip(chains, vals):
                plsc.store_scatter(obuf, [eslot[slot][k], jvec[jl]], val)
            return c

        lax.fori_loop(0, BLK // RB, group, 0)

    for b in range(NBUF):
        gather_start(b, b)

    def outer(i0, carry):
        for b in range(NBUF):
            s = i0 * NBUF + b
            gather_copy(b).wait()

            @pl.when(i0 >= 1)
            def _():
                out_wait(b)

            compute_chunk(s, b)
            for c in out_copies(s, b):
                c.start()

            @pl.when(s + NBUF < SEQ)
            def _():
                gather_start(s + NBUF, b)
        return carry

    lax.fori_loop(0, SEQ // NBUF, outer, 0)
    for t in range(SEQ - SEQ % NBUF, SEQ):
        b = t % NBUF
        gather_copy(b).wait()
        out_wait(b)
        compute_chunk(t, b)
        for c in out_copies(t, b):
            c.start()
    for t in range(SEQ - NBUF, SEQ):
        out_wait(t % NBUF)


def kernel(input_ids, tok_table, pos_table):
    ids_t = input_ids.astype(jnp.int32).T
    mesh = plsc.VectorSubcoreMesh(core_axis_name="c", subcore_axis_name="s")
    out = pl.kernel(
        _body,
        out_type=jax.ShapeDtypeStruct((SEQ, 8, NW, 8, BLK), jnp.float32),
        mesh=mesh,
        compiler_params=pltpu.CompilerParams(use_tc_tiling_on_sc=False,
                                             needs_layout_passes=False),
        scratch_types=[
            pltpu.VMEM((SEQ, BLK), jnp.int32),
            pltpu.VMEM((SEQ, EMB), jnp.float32),
            pltpu.VMEM((NBUF, BLK, EMB), jnp.float32),
            pltpu.VMEM((NBUF * EMB, OPAD), jnp.float32),
        ] + [pltpu.SemaphoreType.DMA] * (2 * NBUF),
    )(ids_t, tok_table, pos_table)
    out = out.transpose(2, 4, 0, 1, 3)
    return out.reshape(BATCH, SEQ, EMB)
